# baseline (speedup 1.0000x reference)
import functools

import jax
import jax.numpy as jnp
from jax import lax
from jax.experimental import pallas as pl
from jax.experimental.pallas import tpu as pltpu
from jax.experimental.pallas import tpu_sc as plsc

_B, _L, _E = 4096, 50, 64
_N = _B * _L
_V = 100000
_NC, _NS = 2, 16
_NW = _NC * _NS
_BPW = _B // _NW
_NB = 4
_NBLK = _BPW // _NB


def _make_kernel():
  mesh = plsc.VectorSubcoreMesh(
      core_axis_name="c", subcore_axis_name="s",
      num_cores=_NC, num_subcores=_NS)

  @functools.partial(
      pl.kernel,
      out_type=(
          jax.ShapeDtypeStruct((_B, _L, _E), jnp.float32),
          jax.ShapeDtypeStruct((_B, _L, _E), jnp.float32),
      ),
      mesh=mesh,
      compiler_params=pltpu.CompilerParams(use_tc_tiling_on_sc=True),
      scratch_types=[
          pltpu.VMEM((_NB * _L,), jnp.int32),
          pltpu.VMEM((_NB * _L, 128), jnp.float32),
          pltpu.VMEM((_NB, _L, _E), jnp.float32),
          pltpu.SemaphoreType.DMA,
      ],
  )
  def emb_kernel(src_idx, tar_idx, src_table, tar_table,
                 src_out, tar_out, idx_v, gbuf, obuf, sem):
    wid = lax.axis_index("s") * _NC + lax.axis_index("c")
    b0 = wid * _BPW

    def run_table(idx_hbm, table_hbm, out_hbm):
      def body(i, _):
        bb = b0 + i * _NB
        pltpu.sync_copy(idx_hbm.at[pl.ds(bb * _L, _NB * _L)], idx_v)
        pltpu.async_copy(table_hbm.at[idx_v], gbuf, sem).wait()
        for j in range(_NB * _L):
          for h in range(_E // 16):
            obuf[j // _L, j % _L, pl.ds(h * 16, 16)] = gbuf[j, pl.ds(h * 16, 16)]
        pltpu.sync_copy(obuf, out_hbm.at[pl.ds(bb, _NB)])
        return 0
      lax.fori_loop(0, _NBLK, body, 0)

    run_table(src_idx, src_table, src_out)
    run_table(tar_idx, tar_table, tar_out)

  return emb_kernel


_EMB = _make_kernel()


@jax.jit
def kernel(src_idx, tar_idx, src_table, tar_table):
  sp = jnp.pad(src_table, ((0, 0), (0, 128 - _E)))
  tp = jnp.pad(tar_table, ((0, 0), (0, 128 - _E)))
  return _EMB(src_idx.reshape(_N), tar_idx.reshape(_N), sp, tp)


# speedup vs baseline: 1.2181x; 1.2181x over previous
import functools

import jax
import jax.numpy as jnp
from jax import lax
from jax.experimental import pallas as pl
from jax.experimental.pallas import tpu as pltpu
from jax.experimental.pallas import tpu_sc as plsc

_B, _L, _E = 4096, 50, 64
_N = _B * _L
_V = 100000
_NC, _NS = 2, 16
_NW = _NC * _NS
_BPW = _B // _NW
_NB = 4
_NBLK = _BPW // _NB
_ROWS = _NB * _L


def _make_kernel():
  mesh = plsc.VectorSubcoreMesh(
      core_axis_name="c", subcore_axis_name="s",
      num_cores=_NC, num_subcores=_NS)

  @functools.partial(
      pl.kernel,
      out_type=(
          jax.ShapeDtypeStruct((_B, _L, _E), jnp.float32),
          jax.ShapeDtypeStruct((_B, _L, _E), jnp.float32),
      ),
      mesh=mesh,
      compiler_params=pltpu.CompilerParams(use_tc_tiling_on_sc=True),
      scratch_types=[
          pltpu.VMEM((_BPW * _L,), jnp.int32),
          pltpu.VMEM((_ROWS, 128), jnp.float32),
          pltpu.VMEM((_ROWS, 128), jnp.float32),
          pltpu.VMEM((_NB, _L, _E), jnp.float32),
          pltpu.VMEM((_NB, _L, _E), jnp.float32),
          pltpu.SemaphoreType.DMA,
          pltpu.SemaphoreType.DMA,
          pltpu.SemaphoreType.DMA,
          pltpu.SemaphoreType.DMA,
      ],
  )
  def emb_kernel(src_idx, tar_idx, src_table, tar_table,
                 src_out, tar_out, idx_all,
                 gbuf0, gbuf1, obuf0, obuf1, g0, g1, o0, o1):
    wid = lax.axis_index("s") * _NC + lax.axis_index("c")
    b0 = wid * _BPW
    gbufs = (gbuf0, gbuf1)
    obufs = (obuf0, obuf1)
    gsems = (g0, g1)
    osems = (o0, o1)

    def run_table(idx_hbm, table_hbm, out_hbm):
      pltpu.sync_copy(idx_hbm.at[pl.ds(b0 * _L, _BPW * _L)], idx_all)

      def gather(k, s):
        pltpu.async_copy(
            table_hbm.at[idx_all.at[pl.ds(k * _ROWS, _ROWS)]],
            gbufs[s], gsems[s])

      def wait_gather(s):
        pltpu.make_async_copy(
            table_hbm.at[pl.ds(0, _ROWS)], gbufs[s], gsems[s]).wait()

      def repack(s):
        gb, ob = gbufs[s], obufs[s]
        for j in range(_ROWS):
          for h in range(_E // 16):
            ob[j // _L, j % _L, pl.ds(h * 16, 16)] = gb[j, pl.ds(h * 16, 16)]

      def store(k, s):
        pltpu.async_copy(obufs[s], out_hbm.at[pl.ds(b0 + k * _NB, _NB)],
                         osems[s])

      def wait_store(s):
        pltpu.make_async_copy(obufs[s], out_hbm.at[pl.ds(b0, _NB)],
                              osems[s]).wait()

      gather(0, 0)
      gather(1, 1)

      def half(i, k, s):
        wait_gather(s)
        pl.when(i > 0)(lambda: wait_store(s))
        repack(s)
        store(k, s)
        pl.when(i < _NBLK // 2 - 1)(lambda: gather(k + 2, s))

      def body(i, _):
        half(i, 2 * i, 0)
        half(i, 2 * i + 1, 1)
        return 0

      lax.fori_loop(0, _NBLK // 2, body, 0)

      wait_store(0)
      wait_store(1)

    run_table(src_idx, src_table, src_out)
    run_table(tar_idx, tar_table, tar_out)

  return emb_kernel


_EMB = _make_kernel()


@jax.jit
def kernel(src_idx, tar_idx, src_table, tar_table):
  sp = jnp.pad(src_table, ((0, 0), (0, 128 - _E)))
  tp = jnp.pad(tar_table, ((0, 0), (0, 128 - _E)))
  return _EMB(src_idx.reshape(_N), tar_idx.reshape(_N), sp, tp)
